# baseline (speedup 1.0000x reference)
import functools

import jax
import jax.numpy as jnp
from jax import lax
from jax.experimental import pallas as pl
from jax.experimental.pallas import tpu as pltpu
from jax.experimental.pallas import tpu_sc as plsc

R0, R1, R2 = 50, 40, 50
N0, N1, N2 = 4, 4, 8
TTR = 32
BATCH = 4096
EMB = N0 * N1 * N2

NC, NS = 2, 16
NW = NC * NS
TPW = BATCH // NW
SEGS = N0 * N1
ROWS_PW = TPW * SEGS

NROWS = R0 * N0 * R1 * N1
NGRP = TPW // 16

_PREC = lax.Precision.HIGHEST


def _mm1_body(a_w, b_w, a_c, b_c, o_w, o_c):
    o_w[...] = jnp.dot(a_w[...], b_w[...], precision=_PREC,
                       preferred_element_type=jnp.float32)
    o_c[...] = jnp.dot(a_c[...], b_c[...], precision=_PREC,
                       preferred_element_type=jnp.float32)


_SC_MESH = plsc.VectorSubcoreMesh(core_axis_name="c", subcore_axis_name="s",
                                  num_cores=NC, num_subcores=NS)


@functools.partial(
    pl.kernel,
    out_type=(jax.ShapeDtypeStruct((NW, EMB, TPW), jnp.float32),
              jax.ShapeDtypeStruct((NW, EMB, TPW), jnp.float32)),
    mesh=_SC_MESH,
    scratch_types=[
        pltpu.VMEM((TPW,), jnp.int32),
        pltpu.VMEM((TPW,), jnp.int32),
        pltpu.VMEM((SEGS, TPW), jnp.int32),
        pltpu.VMEM((ROWS_PW, TTR), jnp.float32),
        pltpu.VMEM((R2, TTR * N2), jnp.float32),
        pltpu.VMEM((TTR * N2 * 16,), jnp.float32),
        pltpu.VMEM((EMB, TPW), jnp.float32),
        pltpu.SemaphoreType.DMA,
    ],
    compiler_params=pltpu.CompilerParams(use_tc_tiling_on_sc=False,
                                         needs_layout_passes=False),
)
def _sc_tt(m1w_hbm, c2tw_hbm, iw_hbm, m1c_hbm, c2tc_hbm, ic_hbm,
           ow_hbm, oc_hbm, idx_v, i2_v, ga_v, a_v, c2t_v, gc_v, out_v, sem):
    wid = lax.axis_index("s") * NC + lax.axis_index("c")
    tok0 = wid * TPW
    lane = lax.iota(jnp.int32, 16)
    c8 = jnp.full((16,), 8, jnp.int32)
    cR1R2 = jnp.full((16,), R1 * R2, jnp.int32)
    cR1 = jnp.full((16,), R1, jnp.int32)
    cR2 = jnp.full((16,), R2, jnp.int32)
    rowpar = lax.div(lane, c8)
    colbase = lax.rem(lane, c8) * SEGS

    for m1_hbm, c2_hbm, i_hbm, o_hbm in (
            (m1w_hbm, c2tw_hbm, iw_hbm, ow_hbm),
            (m1c_hbm, c2tc_hbm, ic_hbm, oc_hbm)):
        pltpu.sync_copy(c2_hbm, c2t_v)
        pltpu.sync_copy(i_hbm.at[pl.ds(tok0, TPW)], idx_v)
        for j in range(NGRP):
            v = idx_v[pl.ds(j * 16, 16)]
            i0 = lax.div(v, cR1R2)
            i1 = lax.rem(lax.div(v, cR2), cR1)
            i2_v[pl.ds(j * 16, 16)] = lax.rem(v, cR2)
            base01 = i0 * (N0 * R1 * N1) + i1 * N1
            row = rowpar + (2 * j)
            for mn in range(SEGS):
                off_mn = (mn // N1) * (R1 * N1) + (mn % N1)
                plsc.store_scatter(ga_v, [row, colbase + mn], base01 + off_mn)
        cps = [pltpu.async_copy(m1_hbm.at[ga_v.at[j]],
                                a_v.at[pl.ds(j * TPW, TPW)], sem)
               for j in range(SEGS)]
        for cp in cps:
            cp.wait()

        def group_body(g, _):
            tvec16 = (g * 16 + lane) * SEGS
            i2g = i2_v[pl.ds(g * 16, 16)]

            @plsc.parallel_loop(0, TTR, unroll=2)
            def _(s):
                s8 = s * N2
                for p in range(N2):
                    val = plsc.load_gather(
                        c2t_v, [i2g, jnp.full((16,), s8 + p, jnp.int32)])
                    gc_v[pl.ds((s8 + p) * 16, 16)] = val

            scol0 = lax.shift_right_logical(i2g, jnp.full((16,), 16, jnp.int32))
            for mnb in range(8):
                rowvs = [tvec16 + (mnb * 2 + k) for k in range(2)]
                zero = jnp.zeros((16,), jnp.float32)

                @plsc.parallel_loop(0, TTR, unroll=2,
                                    carry=((zero,) * 16, scol0))
                def res(si, c):
                    accs, scol = c
                    accs = list(accs)
                    gv = [gc_v[pl.ds(si * (N2 * 16) + p * 16, 16)]
                          for p in range(N2)]
                    for k in range(2):
                        a = plsc.load_gather(a_v, [rowvs[k], scol])
                        for p in range(N2):
                            accs[k * N2 + p] = accs[k * N2 + p] + a * gv[p]
                    return (tuple(accs), scol + 1)

                accs = res[0]
                for k in range(2):
                    for p in range(N2):
                        col = (mnb * 2 + k) * N2 + p
                        out_v[col, pl.ds(g * 16, 16)] = accs[k * N2 + p]
            return 0

        lax.fori_loop(0, NGRP, group_body, 0)
        pltpu.sync_copy(out_v, o_hbm.at[wid])


def kernel(word_indices, context_indices, w_core0, w_core1, w_core2,
           c_core0, c_core1, c_core2):
    c0w = w_core0.reshape(R0 * N0, TTR)
    c1w = w_core1.reshape(TTR, R1 * N1 * TTR)
    c0c = c_core0.reshape(R0 * N0, TTR)
    c1c = c_core1.reshape(TTR, R1 * N1 * TTR)

    m1w, m1c = pl.pallas_call(
        _mm1_body,
        out_shape=(jax.ShapeDtypeStruct((R0 * N0, R1 * N1 * TTR), jnp.float32),
                   jax.ShapeDtypeStruct((R0 * N0, R1 * N1 * TTR), jnp.float32)),
    )(c0w, c1w, c0c, c1c)

    m1w = m1w.reshape(NROWS, TTR)
    m1c = m1c.reshape(NROWS, TTR)

    c2tw = jnp.transpose(w_core2.reshape(TTR, R2, N2), (1, 0, 2)).reshape(
        R2, TTR * N2)
    c2tc = jnp.transpose(c_core2.reshape(TTR, R2, N2), (1, 0, 2)).reshape(
        R2, TTR * N2)

    ow3, oc3 = _sc_tt(m1w, c2tw, word_indices, m1c, c2tc, context_indices)
    ow = jnp.transpose(ow3, (0, 2, 1)).reshape(BATCH, EMB)
    oc = jnp.transpose(oc3, (0, 2, 1)).reshape(BATCH, EMB)
    return ow, oc


# speedup vs baseline: 1.0056x; 1.0056x over previous
import functools

import jax
import jax.numpy as jnp
from jax import lax
from jax.experimental import pallas as pl
from jax.experimental.pallas import tpu as pltpu
from jax.experimental.pallas import tpu_sc as plsc

R0, R1, R2 = 50, 40, 50
N0, N1, N2 = 4, 4, 8
TTR = 32
BATCH = 4096
EMB = N0 * N1 * N2

NC, NS = 2, 16
NW = NC * NS
TPW = BATCH // NW
SEGS = N0 * N1
ROWS_PW = TPW * SEGS

NROWS = R0 * N0 * R1 * N1
NGRP = TPW // 16

_PREC = lax.Precision.HIGHEST


def _mm1_body(a_w, b_w, a_c, b_c, o_w, o_c):
    o_w[...] = jnp.dot(a_w[...], b_w[...], precision=_PREC,
                       preferred_element_type=jnp.float32)
    o_c[...] = jnp.dot(a_c[...], b_c[...], precision=_PREC,
                       preferred_element_type=jnp.float32)


_SC_MESH = plsc.VectorSubcoreMesh(core_axis_name="c", subcore_axis_name="s",
                                  num_cores=NC, num_subcores=NS)


@functools.partial(
    pl.kernel,
    out_type=(jax.ShapeDtypeStruct((NW, EMB, TPW), jnp.float32),
              jax.ShapeDtypeStruct((NW, EMB, TPW), jnp.float32)),
    mesh=_SC_MESH,
    scratch_types=[
        pltpu.VMEM((TPW,), jnp.int32),
        pltpu.VMEM((TPW,), jnp.int32),
        pltpu.VMEM((SEGS, TPW), jnp.int32),
        pltpu.VMEM((ROWS_PW, TTR), jnp.float32),
        pltpu.VMEM((R2, TTR * N2), jnp.float32),
        pltpu.VMEM((TTR * N2 * 16,), jnp.float32),
        pltpu.VMEM((EMB, TPW), jnp.float32),
        pltpu.SemaphoreType.DMA,
    ],
    compiler_params=pltpu.CompilerParams(use_tc_tiling_on_sc=False,
                                         needs_layout_passes=False),
)
def _sc_tt(m1w_hbm, c2tw_hbm, iw_hbm, m1c_hbm, c2tc_hbm, ic_hbm,
           ow_hbm, oc_hbm, idx_v, i2_v, ga_v, a_v, c2t_v, gc_v, out_v, sem):
    wid = lax.axis_index("s") * NC + lax.axis_index("c")
    tok0 = wid * TPW
    lane = lax.iota(jnp.int32, 16)
    c8 = jnp.full((16,), 8, jnp.int32)
    cR1R2 = jnp.full((16,), R1 * R2, jnp.int32)
    cR1 = jnp.full((16,), R1, jnp.int32)
    cR2 = jnp.full((16,), R2, jnp.int32)
    rowpar = lax.div(lane, c8)
    colbase = lax.rem(lane, c8) * SEGS

    for m1_hbm, c2_hbm, i_hbm, o_hbm in (
            (m1w_hbm, c2tw_hbm, iw_hbm, ow_hbm),
            (m1c_hbm, c2tc_hbm, ic_hbm, oc_hbm)):
        pltpu.sync_copy(c2_hbm, c2t_v)
        pltpu.sync_copy(i_hbm.at[pl.ds(tok0, TPW)], idx_v)
        for j in range(NGRP):
            v = idx_v[pl.ds(j * 16, 16)]
            i0 = lax.div(v, cR1R2)
            i1 = lax.rem(lax.div(v, cR2), cR1)
            i2_v[pl.ds(j * 16, 16)] = lax.rem(v, cR2)
            base01 = i0 * (N0 * R1 * N1) + i1 * N1
            row = rowpar + (2 * j)
            for mn in range(SEGS):
                off_mn = (mn // N1) * (R1 * N1) + (mn % N1)
                plsc.store_scatter(ga_v, [row, colbase + mn], base01 + off_mn)
        cps = [pltpu.async_copy(m1_hbm.at[ga_v.at[j]],
                                a_v.at[pl.ds(j * TPW, TPW)], sem)
               for j in range(SEGS)]
        for cp in cps:
            cp.wait()

        def group_body(g, _):
            tvec16 = (g * 16 + lane) * SEGS
            i2g = i2_v[pl.ds(g * 16, 16)]

            @plsc.parallel_loop(0, TTR, unroll=4)
            def _(s):
                s8 = s * N2
                for p in range(N2):
                    val = plsc.load_gather(
                        c2t_v, [i2g, jnp.full((16,), s8 + p, jnp.int32)])
                    gc_v[pl.ds((s8 + p) * 16, 16)] = val

            for mnb in range(8):
                rowvs = [tvec16 + (mnb * 2 + k) for k in range(2)]
                zero = jnp.zeros((16,), jnp.float32)

                @plsc.parallel_loop(0, TTR, unroll=2, carry=(zero,) * 16)
                def accs(si, accs):
                    accs = list(accs)
                    gv = [gc_v[pl.ds(si * (N2 * 16) + p * 16, 16)]
                          for p in range(N2)]
                    scol = jnp.full((16,), si, jnp.int32)
                    for k in range(2):
                        a = plsc.load_gather(a_v, [rowvs[k], scol])
                        for p in range(N2):
                            accs[k * N2 + p] = accs[k * N2 + p] + a * gv[p]
                    return tuple(accs)

                for k in range(2):
                    for p in range(N2):
                        col = (mnb * 2 + k) * N2 + p
                        out_v[col, pl.ds(g * 16, 16)] = accs[k * N2 + p]
            return 0

        lax.fori_loop(0, NGRP, group_body, 0)
        pltpu.sync_copy(out_v, o_hbm.at[wid])


def kernel(word_indices, context_indices, w_core0, w_core1, w_core2,
           c_core0, c_core1, c_core2):
    c0w = w_core0.reshape(R0 * N0, TTR)
    c1w = w_core1.reshape(TTR, R1 * N1 * TTR)
    c0c = c_core0.reshape(R0 * N0, TTR)
    c1c = c_core1.reshape(TTR, R1 * N1 * TTR)

    m1w, m1c = pl.pallas_call(
        _mm1_body,
        out_shape=(jax.ShapeDtypeStruct((R0 * N0, R1 * N1 * TTR), jnp.float32),
                   jax.ShapeDtypeStruct((R0 * N0, R1 * N1 * TTR), jnp.float32)),
    )(c0w, c1w, c0c, c1c)

    m1w = m1w.reshape(NROWS, TTR)
    m1c = m1c.reshape(NROWS, TTR)

    c2tw = jnp.transpose(w_core2.reshape(TTR, R2, N2), (1, 0, 2)).reshape(
        R2, TTR * N2)
    c2tc = jnp.transpose(c_core2.reshape(TTR, R2, N2), (1, 0, 2)).reshape(
        R2, TTR * N2)

    ow3, oc3 = _sc_tt(m1w, c2tw, word_indices, m1c, c2tc, context_indices)
    ow = jnp.transpose(ow3, (0, 2, 1)).reshape(BATCH, EMB)
    oc = jnp.transpose(oc3, (0, 2, 1)).reshape(BATCH, EMB)
    return ow, oc
